# Initial kernel scaffold; baseline (speedup 1.0000x reference)
import jax
import jax.numpy as jnp
from jax.experimental import pallas as pl


def kernel(x, edge_index, batch, W1_0, b1_0, W2_0, b2_0, W1_1, b1_1, W2_1, b2_1, W1_2, b1_2, W2_2, b2_2, Wc, bc):
    raise NotImplementedError("write your pallas kernel here")


# speedup vs baseline: 3.4996x; 3.4996x over previous
import functools

import jax
import jax.numpy as jnp
from jax import lax
from jax.experimental import pallas as pl
from jax.experimental.pallas import tpu as pltpu
from jax.experimental.pallas import tpu_sc as plsc

N = 10000
E = 320000
D = 128
H = 128
C = 10
G = 64

NC = 2
NS = 16
NW = NC * NS
CHUNK = 128
N_CHUNKS = -(-E // (CHUNK * NW))
EPW = N_CHUNKS * CHUNK
E_PAD = EPW * NW
N_PAD = 10240
RPS = N_PAD // NS


def _sc_aggregate(h, src_p, dst_p, zrows):
    mesh = plsc.VectorSubcoreMesh(core_axis_name="c", subcore_axis_name="s")

    @functools.partial(
        pl.kernel,
        mesh=mesh,
        out_type=jax.ShapeDtypeStruct((NC * N_PAD, D), jnp.float32),
        scratch_types=[
            pltpu.VMEM((CHUNK,), jnp.int32),
            pltpu.VMEM((CHUNK,), jnp.int32),
            pltpu.VMEM((CHUNK, D), jnp.float32),
            pltpu.VMEM_SHARED((N_PAD, D), jnp.float32),
            pltpu.SemaphoreType.DMA,
        ],
    )
    def k(h_hbm, src_hbm, dst_hbm, z_hbm, out_hbm, sidx, didx, rows, acc, sem):
        cid = lax.axis_index("c")
        sid = lax.axis_index("s")
        wid = sid * NC + cid

        pltpu.sync_copy(z_hbm, acc.at[pl.ds(sid * RPS, RPS)])
        plsc.subcore_barrier()

        base = wid * EPW

        @pl.loop(0, N_CHUNKS)
        def _(i):
            off = base + i * CHUNK
            pltpu.sync_copy(src_hbm.at[pl.ds(off, CHUNK)], sidx)
            pltpu.sync_copy(dst_hbm.at[pl.ds(off, CHUNK)], didx)
            pltpu.async_copy(h_hbm.at[sidx], rows, sem).wait()
            pltpu.sync_copy(rows, acc.at[didx], add=True)

        plsc.subcore_barrier()
        out_row = cid * N_PAD + sid * RPS
        pltpu.sync_copy(acc.at[pl.ds(sid * RPS, RPS)],
                        out_hbm.at[pl.ds(out_row, RPS)])

    return k(h, src_p, dst_p, zrows)


def _tc_layer(h, agg, Wa, ba, Wb, bb):
    def body(h_ref, a_ref, wa_ref, ba_ref, wb_ref, bb_ref, out_ref):
        s = h_ref[...] + a_ref[0, :N, :] + a_ref[1, :N, :]
        t = jnp.dot(s, wa_ref[...], preferred_element_type=jnp.float32)
        t = jnp.maximum(t + ba_ref[...], 0.0)
        u = jnp.dot(t, wb_ref[...], preferred_element_type=jnp.float32)
        out_ref[...] = jnp.maximum(u + bb_ref[...], 0.0)

    return pl.pallas_call(
        body,
        out_shape=jax.ShapeDtypeStruct((N, H), jnp.float32),
    )(h, agg, Wa, ba.reshape(1, H), Wb, bb.reshape(1, H))


def _tc_final(h, agg, Wa, ba, Wb, bb, batch_t, Wc, bc):
    def body(h_ref, a_ref, wa_ref, ba_ref, wb_ref, bb_ref, bt_ref, wc_ref,
             bc_ref, out_ref):
        s = h_ref[...] + a_ref[0, :N, :] + a_ref[1, :N, :]
        t = jnp.dot(s, wa_ref[...], preferred_element_type=jnp.float32)
        t = jnp.maximum(t + ba_ref[...], 0.0)
        u = jnp.dot(t, wb_ref[...], preferred_element_type=jnp.float32)
        h3 = jnp.maximum(u + bb_ref[...], 0.0)
        seg = (bt_ref[...] == lax.broadcasted_iota(jnp.int32, (G, 1), 0)
               ).astype(jnp.float32)
        sums = jnp.dot(seg, h3, preferred_element_type=jnp.float32)
        counts = jnp.sum(seg, axis=1, keepdims=True)
        pooled = sums / jnp.maximum(counts, 1.0)
        out = jnp.dot(pooled, wc_ref[...], preferred_element_type=jnp.float32)
        out_ref[...] = out + bc_ref[...]

    return pl.pallas_call(
        body,
        out_shape=jax.ShapeDtypeStruct((G, C), jnp.float32),
    )(h, agg, Wa, ba.reshape(1, H), Wb, bb.reshape(1, H), batch_t, Wc,
      bc.reshape(1, C))


def kernel(x, edge_index, batch, W1_0, b1_0, W2_0, b2_0, W1_1, b1_1, W2_1,
           b2_1, W1_2, b1_2, W2_2, b2_2, Wc, bc):
    pad = E_PAD - E
    src_p = jnp.concatenate([edge_index[0], jnp.zeros((pad,), jnp.int32)])
    dst_p = jnp.concatenate(
        [edge_index[1], jnp.full((pad,), N_PAD - 1, jnp.int32)])
    zrows = jnp.zeros((RPS, D), jnp.float32)
    batch_t = batch.reshape(1, N)

    agg = _sc_aggregate(x, src_p, dst_p, zrows)
    agg = agg.reshape(NC, N_PAD, D)
    h = _tc_layer(x, agg, W1_0, b1_0, W2_0, b2_0)
    agg = _sc_aggregate(h, src_p, dst_p, zrows).reshape(NC, N_PAD, D)
    h = _tc_layer(h, agg, W1_1, b1_1, W2_1, b2_1)
    agg = _sc_aggregate(h, src_p, dst_p, zrows).reshape(NC, N_PAD, D)
    return _tc_final(h, agg, W1_2, b1_2, W2_2, b2_2, batch_t, Wc, bc)
